# Initial kernel scaffold; baseline (speedup 1.0000x reference)
import jax
import jax.numpy as jnp
from jax.experimental import pallas as pl


def kernel(x, edge_index, batch, W1, att_src1, att_dst1, b1, W2, att_src2, att_dst2, b2, lin1_w, lin1_b, lin2_w, lin2_b):
    raise NotImplementedError("write your pallas kernel here")


# speedup vs baseline: 1.1349x; 1.1349x over previous
import functools

import jax
import jax.numpy as jnp
from jax.experimental import pallas as pl

N = 10000
E = 320000
F_IN = 128
H1 = 10
C = 64
G = 256

_ROWS = 400


def _mm1_body(x_ref, w_ref, as_ref, ad_ref, h_ref, asrc_ref, adst_ref):
    h = jnp.dot(x_ref[...], w_ref[...], preferred_element_type=jnp.float32)
    h_ref[...] = h
    asrc_ref[...] = jnp.dot(h, as_ref[...], preferred_element_type=jnp.float32)
    adst_ref[...] = jnp.dot(h, ad_ref[...], preferred_element_type=jnp.float32)


def _proj_scores(x, W, att_src, att_dst, heads, ch):
    f_in = x.shape[1]
    eye = jnp.eye(heads, dtype=jnp.float32)
    As = jnp.einsum("hc,hg->hcg", att_src[0], eye).reshape(heads * ch, heads)
    Ad = jnp.einsum("hc,hg->hcg", att_dst[0], eye).reshape(heads * ch, heads)
    grid = (N // _ROWS,)
    h, a_src, a_dst = pl.pallas_call(
        _mm1_body,
        grid=grid,
        in_specs=[
            pl.BlockSpec((_ROWS, f_in), lambda i: (i, 0)),
            pl.BlockSpec((f_in, heads * ch), lambda i: (0, 0)),
            pl.BlockSpec((heads * ch, heads), lambda i: (0, 0)),
            pl.BlockSpec((heads * ch, heads), lambda i: (0, 0)),
        ],
        out_specs=[
            pl.BlockSpec((_ROWS, heads * ch), lambda i: (i, 0)),
            pl.BlockSpec((_ROWS, heads), lambda i: (i, 0)),
            pl.BlockSpec((_ROWS, heads), lambda i: (i, 0)),
        ],
        out_shape=[
            jax.ShapeDtypeStruct((N, heads * ch), jnp.float32),
            jax.ShapeDtypeStruct((N, heads), jnp.float32),
            jax.ShapeDtypeStruct((N, heads), jnp.float32),
        ],
    )(x, W, As, Ad)
    return h, a_src, a_dst


def _edge_aggregate(h, a_src, a_dst, src, dst, heads, ch):
    alpha = a_src[src] + a_dst[dst]
    w = jnp.exp(jnp.where(alpha >= 0, alpha, 0.2 * alpha))
    den = jax.ops.segment_sum(w, dst, num_segments=N)
    msg = h.reshape(N, heads, ch)[src] * w[:, :, None]
    num = jax.ops.segment_sum(msg, dst, num_segments=N)
    return (num / (den[:, :, None] + 1e-16)).reshape(N, heads * ch)


def _elu(x):
    return jnp.where(x > 0, x, jnp.expm1(jnp.minimum(x, 0.0)))


def kernel(x, edge_index, batch, W1, att_src1, att_dst1, b1,
           W2, att_src2, att_dst2, b2, lin1_w, lin1_b, lin2_w, lin2_b):
    src = edge_index[0]
    dst = edge_index[1]

    h1, as1, ad1 = _proj_scores(x, W1, att_src1, att_dst1, H1, C)
    x1 = _elu(_edge_aggregate(h1, as1, ad1, src, dst, H1, C) + b1)

    h2, as2, ad2 = _proj_scores(x1, W2, att_src2, att_dst2, 1, C)
    x2 = _elu(_edge_aggregate(h2, as2, ad2, src, dst, 1, C) + b2)

    sums = jax.ops.segment_sum(x2, batch, num_segments=G)
    counts = jax.ops.segment_sum(jnp.ones((N,), jnp.float32), batch,
                                 num_segments=G)
    pooled = sums / jnp.maximum(counts, 1.0)[:, None]
    hid = jax.nn.relu(pooled @ lin1_w + lin1_b)
    return hid @ lin2_w + lin2_b


# speedup vs baseline: 8.6333x; 7.6070x over previous
import functools

import jax
import jax.numpy as jnp
from jax import lax
from jax.experimental import pallas as pl
from jax.experimental.pallas import tpu as pltpu
from jax.experimental.pallas import tpu_sc as plsc

N = 10000
E = 320000
F_IN = 128
H1 = 10
C = 64
G = 256

_NC = 2
_NS = 16
_NW = _NC * _NS
_CE = 4000
_NCH = E // _CE
_ROWS = 400


def _elu(v):
    return jnp.where(v > 0, v, jnp.exp(jnp.minimum(v, 0.0)) - 1.0)


def _mm1_body(x_ref, w_ref, as_ref, ad_ref, h_ref, asrc_ref, adst_ref):
    h = jnp.dot(x_ref[...], w_ref[...], preferred_element_type=jnp.float32)
    h_ref[...] = h
    asrc_ref[...] = jnp.dot(h, as_ref[...], preferred_element_type=jnp.float32)
    adst_ref[...] = jnp.dot(h, ad_ref[...], preferred_element_type=jnp.float32)


def _proj_scores(x, W, att_src, att_dst, heads, ch):
    f_in = x.shape[1]
    eye = jnp.eye(heads, dtype=jnp.float32)
    As = jnp.einsum("hc,hg->hcg", att_src[0], eye).reshape(heads * ch, heads)
    Ad = jnp.einsum("hc,hg->hcg", att_dst[0], eye).reshape(heads * ch, heads)
    h, a_src, a_dst = pl.pallas_call(
        _mm1_body,
        grid=(N // _ROWS,),
        in_specs=[
            pl.BlockSpec((_ROWS, f_in), lambda i: (i, 0)),
            pl.BlockSpec((f_in, heads * ch), lambda i: (0, 0)),
            pl.BlockSpec((heads * ch, heads), lambda i: (0, 0)),
            pl.BlockSpec((heads * ch, heads), lambda i: (0, 0)),
        ],
        out_specs=[
            pl.BlockSpec((_ROWS, heads * ch), lambda i: (i, 0)),
            pl.BlockSpec((_ROWS, heads), lambda i: (i, 0)),
            pl.BlockSpec((_ROWS, heads), lambda i: (i, 0)),
        ],
        out_shape=[
            jax.ShapeDtypeStruct((N, heads * ch), jnp.float32),
            jax.ShapeDtypeStruct((N, heads), jnp.float32),
            jax.ShapeDtypeStruct((N, heads), jnp.float32),
        ],
    )(x, W, As, Ad)
    return h, a_src, a_dst


def _sc_edge_body(cpj, njob, jph, h_hbm, asrc_hbm, adst_hbm, src_hbm,
                  dst_hbm, zeros_hbm, out_hbm, hs_v, num_v, asrc_v, adst_v,
                  sb_v, db_v):
    cc = lax.axis_index("c")
    ss = lax.axis_index("s")
    wid = ss * _NC + cc
    nt = njob // _NW

    for t in range(nt):
        job = wid + t * _NW
        head = job // jph
        pltpu.sync_copy(h_hbm.at[pl.ds(pl.multiple_of(job * cpj * N, 8),
                                       cpj * N)], hs_v)
        pltpu.sync_copy(asrc_hbm.at[pl.ds(pl.multiple_of(head * N, 8), N)],
                        asrc_v)
        pltpu.sync_copy(adst_hbm.at[pl.ds(pl.multiple_of(head * N, 8), N)],
                        adst_v)
        pltpu.sync_copy(zeros_hbm, num_v)

        def chunk(ch, _):
            eo = pl.multiple_of(ch * _CE, 8)
            pltpu.sync_copy(src_hbm.at[pl.ds(eo, _CE)], sb_v)
            pltpu.sync_copy(dst_hbm.at[pl.ds(eo, _CE)], db_v)

            def group(g, _):
                go = pl.multiple_of(g * 16, 8)
                src16 = sb_v[pl.ds(go, 16)]
                dst16 = db_v[pl.ds(go, 16)]
                a = (plsc.load_gather(asrc_v, [src16])
                     + plsc.load_gather(adst_v, [dst16]))
                w16 = jnp.exp(jnp.where(a >= 0, a, 0.2 * a))
                plsc.addupdate_scatter(num_v, [dst16 + cpj * N], w16)
                for col in range(cpj):
                    vals = plsc.load_gather(hs_v, [src16 + col * N])
                    plsc.addupdate_scatter(num_v, [dst16 + col * N],
                                           vals * w16)
                return _
            lax.fori_loop(0, _CE // 16, group, None)
            return _
        lax.fori_loop(0, _NCH, chunk, None)

        obase = pl.multiple_of(job * (cpj + 1) * N, 8)
        pltpu.sync_copy(num_v, out_hbm.at[pl.ds(obase, (cpj + 1) * N)])


def _sc_edge(hT, asrc, adst, src, dst, cpj, njob, jph):
    mesh = plsc.VectorSubcoreMesh(core_axis_name="c", subcore_axis_name="s")
    zeros = jnp.zeros(((cpj + 1) * N,), jnp.float32)
    fn = pl.kernel(
        functools.partial(_sc_edge_body, cpj, njob, jph),
        out_type=jax.ShapeDtypeStruct((njob * (cpj + 1) * N,), jnp.float32),
        mesh=mesh,
        compiler_params=pltpu.CompilerParams(needs_layout_passes=False),
        scratch_types=[
            pltpu.VMEM((cpj * N,), jnp.float32),
            pltpu.VMEM(((cpj + 1) * N,), jnp.float32),
            pltpu.VMEM((N,), jnp.float32),
            pltpu.VMEM((N,), jnp.float32),
            pltpu.VMEM((_CE,), jnp.int32),
            pltpu.VMEM((_CE,), jnp.int32),
        ],
    )
    return fn(hT, asrc, adst, src, dst, zeros)


def _k2_body(num_ref, den_ref, b1_ref, w2_ref, as_ref, ad_ref,
             h2_ref, a2s_ref, a2d_ref):
    rows = num_ref.shape[0]
    den = den_ref[...].reshape(rows, H1, 1) + 1e-16
    den = jnp.broadcast_to(den, (rows, H1, C)).reshape(rows, H1 * C)
    x1 = _elu(num_ref[...] / den + b1_ref[...])
    h2 = jnp.dot(x1, w2_ref[...], preferred_element_type=jnp.float32)
    h2_ref[...] = h2
    a2s_ref[...] = jnp.dot(h2, as_ref[...], preferred_element_type=jnp.float32)
    a2d_ref[...] = jnp.dot(h2, ad_ref[...], preferred_element_type=jnp.float32)


def _combine_l1(num_t, den_t, b1, W2, att_src2, att_dst2):
    h2, a2s, a2d = pl.pallas_call(
        _k2_body,
        grid=(N // _ROWS,),
        in_specs=[
            pl.BlockSpec((_ROWS, H1 * C), lambda i: (i, 0)),
            pl.BlockSpec((_ROWS, H1), lambda i: (i, 0)),
            pl.BlockSpec((1, H1 * C), lambda i: (0, 0)),
            pl.BlockSpec((H1 * C, C), lambda i: (0, 0)),
            pl.BlockSpec((C, 1), lambda i: (0, 0)),
            pl.BlockSpec((C, 1), lambda i: (0, 0)),
        ],
        out_specs=[
            pl.BlockSpec((_ROWS, C), lambda i: (i, 0)),
            pl.BlockSpec((_ROWS, 1), lambda i: (i, 0)),
            pl.BlockSpec((_ROWS, 1), lambda i: (i, 0)),
        ],
        out_shape=[
            jax.ShapeDtypeStruct((N, C), jnp.float32),
            jax.ShapeDtypeStruct((N, 1), jnp.float32),
            jax.ShapeDtypeStruct((N, 1), jnp.float32),
        ],
    )(num_t, den_t, b1.reshape(1, H1 * C), W2,
      att_src2[0].reshape(C, 1), att_dst2[0].reshape(C, 1))
    return h2, a2s, a2d


def _k3_body(num_ref, den_ref, b2_ref, batch_ref, l1w_ref, l1b_ref,
             l2w_ref, l2b_ref, out_ref, pool_ref):
    i = pl.program_id(0)
    nblk = pl.num_programs(0)
    rows = num_ref.shape[0]

    @pl.when(i == 0)
    def _():
        pool_ref[...] = jnp.zeros_like(pool_ref)

    x2 = _elu(num_ref[...] / (den_ref[...] + 1e-16) + b2_ref[...])
    ext = lax.concatenate([x2, jnp.ones((rows, C), jnp.float32)], 1)
    onehot = (batch_ref[...] ==
              lax.broadcasted_iota(jnp.int32, (rows, G), 1).astype(jnp.float32)
              ).astype(jnp.float32)
    pool_ref[...] += lax.dot_general(
        onehot, ext, (((0,), (0,)), ((), ())),
        preferred_element_type=jnp.float32)

    @pl.when(i == nblk - 1)
    def _():
        pooled = pool_ref[...]
        mean = pooled[:, :C] / jnp.maximum(pooled[:, C:C + 1], 1.0)
        hid = jnp.maximum(
            jnp.dot(mean, l1w_ref[...], preferred_element_type=jnp.float32)
            + l1b_ref[...], 0.0)
        out_ref[...] = (jnp.dot(hid, l2w_ref[...],
                                preferred_element_type=jnp.float32)
                        + l2b_ref[...])


def _pool_mlp(num2, den2, b2, batch_f, lin1_w, lin1_b, lin2_w, lin2_b):
    nhid = lin1_w.shape[1]
    nout = lin2_w.shape[1]
    return pl.pallas_call(
        _k3_body,
        grid=(N // _ROWS,),
        in_specs=[
            pl.BlockSpec((_ROWS, C), lambda i: (i, 0)),
            pl.BlockSpec((_ROWS, 1), lambda i: (i, 0)),
            pl.BlockSpec((1, C), lambda i: (0, 0)),
            pl.BlockSpec((_ROWS, 1), lambda i: (i, 0)),
            pl.BlockSpec((C, nhid), lambda i: (0, 0)),
            pl.BlockSpec((1, nhid), lambda i: (0, 0)),
            pl.BlockSpec((nhid, nout), lambda i: (0, 0)),
            pl.BlockSpec((1, nout), lambda i: (0, 0)),
        ],
        out_specs=pl.BlockSpec((G, nout), lambda i: (0, 0)),
        out_shape=jax.ShapeDtypeStruct((G, nout), jnp.float32),
        scratch_shapes=[pltpu.VMEM((G, 2 * C), jnp.float32)],
    )(num2, den2, b2.reshape(1, C), batch_f, lin1_w,
      lin1_b.reshape(1, nhid), lin2_w, lin2_b.reshape(1, nout))


def kernel(x, edge_index, batch, W1, att_src1, att_dst1, b1,
           W2, att_src2, att_dst2, b2, lin1_w, lin1_b, lin2_w, lin2_b):
    src = edge_index[0]
    dst = edge_index[1]
    batch_f = batch.astype(jnp.float32).reshape(N, 1)

    h1, as1, ad1 = _proj_scores(x, W1, att_src1, att_dst1, H1, C)
    h1T = h1.T.reshape(H1 * C * N)
    as1t = as1.T.reshape(H1 * N)
    ad1t = ad1.T.reshape(H1 * N)
    njob1 = H1 * C // 4
    out1 = _sc_edge(h1T, as1t, ad1t, src, dst, 4, njob1, C // 4)
    o1 = out1.reshape(njob1, 5, N)
    num1_t = o1[:, :4, :].reshape(H1 * C, N).T
    den1_t = o1[::(C // 4), 4, :].T

    h2, a2s, a2d = _combine_l1(num1_t, den1_t, b1, W2, att_src2, att_dst2)

    h2T = h2.T.reshape(C * N)
    njob2 = C // 2
    out2 = _sc_edge(h2T, a2s.reshape(N), a2d.reshape(N), src, dst,
                    2, njob2, njob2)
    o2 = out2.reshape(njob2, 3, N)
    num2_t = o2[:, :2, :].reshape(C, N).T
    den2_t = o2[0, 2, :].reshape(N, 1)

    return _pool_mlp(num2_t, den2_t, b2, batch_f,
                     lin1_w, lin1_b, lin2_w, lin2_b)


# speedup vs baseline: 8.7376x; 1.0121x over previous
import functools

import jax
import jax.numpy as jnp
from jax import lax
from jax.experimental import pallas as pl
from jax.experimental.pallas import tpu as pltpu
from jax.experimental.pallas import tpu_sc as plsc

N = 10000
E = 320000
F_IN = 128
H1 = 10
C = 64
G = 256

_NC = 2
_NS = 16
_NW = _NC * _NS
_CE = 4000
_NCH = E // _CE
_ROWS = 400


def _elu(v):
    return jnp.where(v > 0, v, jnp.exp(jnp.minimum(v, 0.0)) - 1.0)


def _mm1_body(x_ref, w_ref, as_ref, ad_ref, h_ref, asrc_ref, adst_ref):
    h = jnp.dot(x_ref[...], w_ref[...], preferred_element_type=jnp.float32)
    h_ref[...] = h
    asrc_ref[...] = jnp.dot(h, as_ref[...], preferred_element_type=jnp.float32)
    adst_ref[...] = jnp.dot(h, ad_ref[...], preferred_element_type=jnp.float32)


def _proj_scores(x, W, att_src, att_dst, heads, ch):
    f_in = x.shape[1]
    eye = jnp.eye(heads, dtype=jnp.float32)
    As = jnp.einsum("hc,hg->hcg", att_src[0], eye).reshape(heads * ch, heads)
    Ad = jnp.einsum("hc,hg->hcg", att_dst[0], eye).reshape(heads * ch, heads)
    h, a_src, a_dst = pl.pallas_call(
        _mm1_body,
        grid=(N // _ROWS,),
        in_specs=[
            pl.BlockSpec((_ROWS, f_in), lambda i: (i, 0)),
            pl.BlockSpec((f_in, heads * ch), lambda i: (0, 0)),
            pl.BlockSpec((heads * ch, heads), lambda i: (0, 0)),
            pl.BlockSpec((heads * ch, heads), lambda i: (0, 0)),
        ],
        out_specs=[
            pl.BlockSpec((_ROWS, heads * ch), lambda i: (i, 0)),
            pl.BlockSpec((_ROWS, heads), lambda i: (i, 0)),
            pl.BlockSpec((_ROWS, heads), lambda i: (i, 0)),
        ],
        out_shape=[
            jax.ShapeDtypeStruct((N, heads * ch), jnp.float32),
            jax.ShapeDtypeStruct((N, heads), jnp.float32),
            jax.ShapeDtypeStruct((N, heads), jnp.float32),
        ],
    )(x, W, As, Ad)
    return h, a_src, a_dst


def _sc_edge_body(cpj, njob, jph, h_hbm, asrc_hbm, adst_hbm, src_hbm,
                  dst_hbm, zeros_hbm, out_hbm, hs_v, num_v, asrc_v, adst_v,
                  sb_v, db_v):
    cc = lax.axis_index("c")
    ss = lax.axis_index("s")
    wid = ss * _NC + cc
    nt = njob // _NW

    for t in range(nt):
        job = wid + t * _NW
        head = job // jph
        pltpu.sync_copy(h_hbm.at[pl.ds(pl.multiple_of(job * cpj * N, 8),
                                       cpj * N)], hs_v)
        pltpu.sync_copy(asrc_hbm.at[pl.ds(pl.multiple_of(head * N, 8), N)],
                        asrc_v)
        pltpu.sync_copy(adst_hbm.at[pl.ds(pl.multiple_of(head * N, 8), N)],
                        adst_v)
        pltpu.sync_copy(zeros_hbm, num_v)

        def chunk(ch, _):
            eo = pl.multiple_of(ch * _CE, 8)
            pltpu.sync_copy(src_hbm.at[pl.ds(eo, _CE)], sb_v)
            pltpu.sync_copy(dst_hbm.at[pl.ds(eo, _CE)], db_v)

            def group(g, _):
                go = pl.multiple_of(g * 16, 8)
                src16 = sb_v[pl.ds(go, 16)]
                dst16 = db_v[pl.ds(go, 16)]
                a = (plsc.load_gather(asrc_v, [src16])
                     + plsc.load_gather(adst_v, [dst16]))
                w16 = jnp.exp(jnp.where(a >= 0, a, 0.2 * a))
                plsc.addupdate_scatter(num_v, [dst16 + cpj * N], w16)
                for col in range(cpj):
                    vals = plsc.load_gather(hs_v, [src16 + col * N])
                    plsc.addupdate_scatter(num_v, [dst16 + col * N],
                                           vals * w16)
                return _
            lax.fori_loop(0, _CE // 16, group, None, unroll=8)
            return _
        lax.fori_loop(0, _NCH, chunk, None)

        obase = pl.multiple_of(job * (cpj + 1) * N, 8)
        pltpu.sync_copy(num_v, out_hbm.at[pl.ds(obase, (cpj + 1) * N)])


def _sc_edge(hT, asrc, adst, src, dst, cpj, njob, jph):
    mesh = plsc.VectorSubcoreMesh(core_axis_name="c", subcore_axis_name="s")
    zeros = jnp.zeros(((cpj + 1) * N,), jnp.float32)
    fn = pl.kernel(
        functools.partial(_sc_edge_body, cpj, njob, jph),
        out_type=jax.ShapeDtypeStruct((njob * (cpj + 1) * N,), jnp.float32),
        mesh=mesh,
        compiler_params=pltpu.CompilerParams(needs_layout_passes=False),
        scratch_types=[
            pltpu.VMEM((cpj * N,), jnp.float32),
            pltpu.VMEM(((cpj + 1) * N,), jnp.float32),
            pltpu.VMEM((N,), jnp.float32),
            pltpu.VMEM((N,), jnp.float32),
            pltpu.VMEM((_CE,), jnp.int32),
            pltpu.VMEM((_CE,), jnp.int32),
        ],
    )
    return fn(hT, asrc, adst, src, dst, zeros)


def _k2_body(num_ref, den_ref, b1_ref, w2_ref, as_ref, ad_ref,
             h2_ref, a2s_ref, a2d_ref):
    rows = num_ref.shape[0]
    den = den_ref[...].reshape(rows, H1, 1) + 1e-16
    den = jnp.broadcast_to(den, (rows, H1, C)).reshape(rows, H1 * C)
    x1 = _elu(num_ref[...] / den + b1_ref[...])
    h2 = jnp.dot(x1, w2_ref[...], preferred_element_type=jnp.float32)
    h2_ref[...] = h2
    a2s_ref[...] = jnp.dot(h2, as_ref[...], preferred_element_type=jnp.float32)
    a2d_ref[...] = jnp.dot(h2, ad_ref[...], preferred_element_type=jnp.float32)


def _combine_l1(num_t, den_t, b1, W2, att_src2, att_dst2):
    h2, a2s, a2d = pl.pallas_call(
        _k2_body,
        grid=(N // _ROWS,),
        in_specs=[
            pl.BlockSpec((_ROWS, H1 * C), lambda i: (i, 0)),
            pl.BlockSpec((_ROWS, H1), lambda i: (i, 0)),
            pl.BlockSpec((1, H1 * C), lambda i: (0, 0)),
            pl.BlockSpec((H1 * C, C), lambda i: (0, 0)),
            pl.BlockSpec((C, 1), lambda i: (0, 0)),
            pl.BlockSpec((C, 1), lambda i: (0, 0)),
        ],
        out_specs=[
            pl.BlockSpec((_ROWS, C), lambda i: (i, 0)),
            pl.BlockSpec((_ROWS, 1), lambda i: (i, 0)),
            pl.BlockSpec((_ROWS, 1), lambda i: (i, 0)),
        ],
        out_shape=[
            jax.ShapeDtypeStruct((N, C), jnp.float32),
            jax.ShapeDtypeStruct((N, 1), jnp.float32),
            jax.ShapeDtypeStruct((N, 1), jnp.float32),
        ],
    )(num_t, den_t, b1.reshape(1, H1 * C), W2,
      att_src2[0].reshape(C, 1), att_dst2[0].reshape(C, 1))
    return h2, a2s, a2d


def _k3_body(num_ref, den_ref, b2_ref, batch_ref, l1w_ref, l1b_ref,
             l2w_ref, l2b_ref, out_ref, pool_ref):
    i = pl.program_id(0)
    nblk = pl.num_programs(0)
    rows = num_ref.shape[0]

    @pl.when(i == 0)
    def _():
        pool_ref[...] = jnp.zeros_like(pool_ref)

    x2 = _elu(num_ref[...] / (den_ref[...] + 1e-16) + b2_ref[...])
    ext = lax.concatenate([x2, jnp.ones((rows, C), jnp.float32)], 1)
    onehot = (batch_ref[...] ==
              lax.broadcasted_iota(jnp.int32, (rows, G), 1).astype(jnp.float32)
              ).astype(jnp.float32)
    pool_ref[...] += lax.dot_general(
        onehot, ext, (((0,), (0,)), ((), ())),
        preferred_element_type=jnp.float32)

    @pl.when(i == nblk - 1)
    def _():
        pooled = pool_ref[...]
        mean = pooled[:, :C] / jnp.maximum(pooled[:, C:C + 1], 1.0)
        hid = jnp.maximum(
            jnp.dot(mean, l1w_ref[...], preferred_element_type=jnp.float32)
            + l1b_ref[...], 0.0)
        out_ref[...] = (jnp.dot(hid, l2w_ref[...],
                                preferred_element_type=jnp.float32)
                        + l2b_ref[...])


def _pool_mlp(num2, den2, b2, batch_f, lin1_w, lin1_b, lin2_w, lin2_b):
    nhid = lin1_w.shape[1]
    nout = lin2_w.shape[1]
    return pl.pallas_call(
        _k3_body,
        grid=(N // _ROWS,),
        in_specs=[
            pl.BlockSpec((_ROWS, C), lambda i: (i, 0)),
            pl.BlockSpec((_ROWS, 1), lambda i: (i, 0)),
            pl.BlockSpec((1, C), lambda i: (0, 0)),
            pl.BlockSpec((_ROWS, 1), lambda i: (i, 0)),
            pl.BlockSpec((C, nhid), lambda i: (0, 0)),
            pl.BlockSpec((1, nhid), lambda i: (0, 0)),
            pl.BlockSpec((nhid, nout), lambda i: (0, 0)),
            pl.BlockSpec((1, nout), lambda i: (0, 0)),
        ],
        out_specs=pl.BlockSpec((G, nout), lambda i: (0, 0)),
        out_shape=jax.ShapeDtypeStruct((G, nout), jnp.float32),
        scratch_shapes=[pltpu.VMEM((G, 2 * C), jnp.float32)],
    )(num2, den2, b2.reshape(1, C), batch_f, lin1_w,
      lin1_b.reshape(1, nhid), lin2_w, lin2_b.reshape(1, nout))


def kernel(x, edge_index, batch, W1, att_src1, att_dst1, b1,
           W2, att_src2, att_dst2, b2, lin1_w, lin1_b, lin2_w, lin2_b):
    src = edge_index[0]
    dst = edge_index[1]
    batch_f = batch.astype(jnp.float32).reshape(N, 1)

    h1, as1, ad1 = _proj_scores(x, W1, att_src1, att_dst1, H1, C)
    h1T = h1.T.reshape(H1 * C * N)
    as1t = as1.T.reshape(H1 * N)
    ad1t = ad1.T.reshape(H1 * N)
    njob1 = H1 * C // 4
    out1 = _sc_edge(h1T, as1t, ad1t, src, dst, 4, njob1, C // 4)
    o1 = out1.reshape(njob1, 5, N)
    num1_t = o1[:, :4, :].reshape(H1 * C, N).T
    den1_t = o1[::(C // 4), 4, :].T

    h2, a2s, a2d = _combine_l1(num1_t, den1_t, b1, W2, att_src2, att_dst2)

    h2T = h2.T.reshape(C * N)
    njob2 = C // 2
    out2 = _sc_edge(h2T, a2s.reshape(N), a2d.reshape(N), src, dst,
                    2, njob2, njob2)
    o2 = out2.reshape(njob2, 3, N)
    num2_t = o2[:, :2, :].reshape(C, N).T
    den2_t = o2[0, 2, :].reshape(N, 1)

    return _pool_mlp(num2_t, den2_t, b2, batch_f,
                     lin1_w, lin1_b, lin2_w, lin2_b)


# speedup vs baseline: 9.2434x; 1.0579x over previous
import functools

import jax
import jax.numpy as jnp
from jax import lax
from jax.experimental import pallas as pl
from jax.experimental.pallas import tpu as pltpu
from jax.experimental.pallas import tpu_sc as plsc

N = 10000
E = 320000
F_IN = 128
H1 = 10
C = 64
G = 256

_NC = 2
_NS = 16
_NW = _NC * _NS
_CE = 10000
_NCH = E // _CE
_ROWS = 400


def _elu(v):
    return jnp.where(v > 0, v, jnp.exp(jnp.minimum(v, 0.0)) - 1.0)


def _mm1_body(x_ref, w_ref, as_ref, ad_ref, h_ref, asrc_ref, adst_ref):
    h = jnp.dot(x_ref[...], w_ref[...], preferred_element_type=jnp.float32)
    h_ref[...] = h
    asrc_ref[...] = jnp.dot(h, as_ref[...], preferred_element_type=jnp.float32)
    adst_ref[...] = jnp.dot(h, ad_ref[...], preferred_element_type=jnp.float32)


def _proj_scores(x, W, att_src, att_dst, heads, ch):
    f_in = x.shape[1]
    eye = jnp.eye(heads, dtype=jnp.float32)
    As = jnp.einsum("hc,hg->hcg", att_src[0], eye).reshape(heads * ch, heads)
    Ad = jnp.einsum("hc,hg->hcg", att_dst[0], eye).reshape(heads * ch, heads)
    h, a_src, a_dst = pl.pallas_call(
        _mm1_body,
        grid=(N // _ROWS,),
        in_specs=[
            pl.BlockSpec((_ROWS, f_in), lambda i: (i, 0)),
            pl.BlockSpec((f_in, heads * ch), lambda i: (0, 0)),
            pl.BlockSpec((heads * ch, heads), lambda i: (0, 0)),
            pl.BlockSpec((heads * ch, heads), lambda i: (0, 0)),
        ],
        out_specs=[
            pl.BlockSpec((_ROWS, heads * ch), lambda i: (i, 0)),
            pl.BlockSpec((_ROWS, heads), lambda i: (i, 0)),
            pl.BlockSpec((_ROWS, heads), lambda i: (i, 0)),
        ],
        out_shape=[
            jax.ShapeDtypeStruct((N, heads * ch), jnp.float32),
            jax.ShapeDtypeStruct((N, heads), jnp.float32),
            jax.ShapeDtypeStruct((N, heads), jnp.float32),
        ],
    )(x, W, As, Ad)
    return h, a_src, a_dst


def _sc_edge_body(cpj, njob, jph, h_hbm, asrc_hbm, adst_hbm, src_hbm,
                  dst_hbm, zeros_hbm, out_hbm, hs_v, num_v, asrc_v, adst_v,
                  sb_v, db_v):
    cc = lax.axis_index("c")
    ss = lax.axis_index("s")
    wid = ss * _NC + cc
    nt = njob // _NW

    for t in range(nt):
        job = wid + t * _NW
        head = job // jph
        pltpu.sync_copy(h_hbm.at[pl.ds(pl.multiple_of(job * cpj * N, 8),
                                       cpj * N)], hs_v)
        pltpu.sync_copy(asrc_hbm.at[pl.ds(pl.multiple_of(head * N, 8), N)],
                        asrc_v)
        pltpu.sync_copy(adst_hbm.at[pl.ds(pl.multiple_of(head * N, 8), N)],
                        adst_v)
        pltpu.sync_copy(zeros_hbm, num_v)

        def chunk(ch, _):
            eo = pl.multiple_of(ch * _CE, 8)
            pltpu.sync_copy(src_hbm.at[pl.ds(eo, _CE)], sb_v)
            pltpu.sync_copy(dst_hbm.at[pl.ds(eo, _CE)], db_v)

            def group(g, _):
                go = pl.multiple_of(g * 16, 8)
                src16 = sb_v[pl.ds(go, 16)]
                dst16 = db_v[pl.ds(go, 16)]
                a = (plsc.load_gather(asrc_v, [src16])
                     + plsc.load_gather(adst_v, [dst16]))
                w16 = jnp.exp(jnp.where(a >= 0, a, 0.2 * a))

                @pl.when(job % jph == 0)
                def _():
                    plsc.addupdate_scatter(num_v, [dst16 + cpj * N], w16)
                for col in range(cpj):
                    vals = plsc.load_gather(hs_v, [src16 + col * N])
                    plsc.addupdate_scatter(num_v, [dst16 + col * N],
                                           vals * w16)
                return _
            lax.fori_loop(0, _CE // 16, group, None, unroll=8)
            return _
        lax.fori_loop(0, _NCH, chunk, None)

        obase = pl.multiple_of(job * (cpj + 1) * N, 8)
        pltpu.sync_copy(num_v, out_hbm.at[pl.ds(obase, (cpj + 1) * N)])


def _sc_edge(hT, asrc, adst, src, dst, cpj, njob, jph):
    mesh = plsc.VectorSubcoreMesh(core_axis_name="c", subcore_axis_name="s")
    zeros = jnp.zeros(((cpj + 1) * N,), jnp.float32)
    fn = pl.kernel(
        functools.partial(_sc_edge_body, cpj, njob, jph),
        out_type=jax.ShapeDtypeStruct((njob * (cpj + 1) * N,), jnp.float32),
        mesh=mesh,
        compiler_params=pltpu.CompilerParams(needs_layout_passes=False),
        scratch_types=[
            pltpu.VMEM((cpj * N,), jnp.float32),
            pltpu.VMEM(((cpj + 1) * N,), jnp.float32),
            pltpu.VMEM((N,), jnp.float32),
            pltpu.VMEM((N,), jnp.float32),
            pltpu.VMEM((_CE,), jnp.int32),
            pltpu.VMEM((_CE,), jnp.int32),
        ],
    )
    return fn(hT, asrc, adst, src, dst, zeros)


def _k2_body(num_ref, den_ref, b1_ref, w2_ref, as_ref, ad_ref,
             h2_ref, a2s_ref, a2d_ref):
    rows = num_ref.shape[0]
    den = den_ref[...].reshape(rows, H1, 1) + 1e-16
    den = jnp.broadcast_to(den, (rows, H1, C)).reshape(rows, H1 * C)
    x1 = _elu(num_ref[...] / den + b1_ref[...])
    h2 = jnp.dot(x1, w2_ref[...], preferred_element_type=jnp.float32)
    h2_ref[...] = h2
    a2s_ref[...] = jnp.dot(h2, as_ref[...], preferred_element_type=jnp.float32)
    a2d_ref[...] = jnp.dot(h2, ad_ref[...], preferred_element_type=jnp.float32)


def _combine_l1(num_t, den_t, b1, W2, att_src2, att_dst2):
    h2, a2s, a2d = pl.pallas_call(
        _k2_body,
        grid=(N // _ROWS,),
        in_specs=[
            pl.BlockSpec((_ROWS, H1 * C), lambda i: (i, 0)),
            pl.BlockSpec((_ROWS, H1), lambda i: (i, 0)),
            pl.BlockSpec((1, H1 * C), lambda i: (0, 0)),
            pl.BlockSpec((H1 * C, C), lambda i: (0, 0)),
            pl.BlockSpec((C, 1), lambda i: (0, 0)),
            pl.BlockSpec((C, 1), lambda i: (0, 0)),
        ],
        out_specs=[
            pl.BlockSpec((_ROWS, C), lambda i: (i, 0)),
            pl.BlockSpec((_ROWS, 1), lambda i: (i, 0)),
            pl.BlockSpec((_ROWS, 1), lambda i: (i, 0)),
        ],
        out_shape=[
            jax.ShapeDtypeStruct((N, C), jnp.float32),
            jax.ShapeDtypeStruct((N, 1), jnp.float32),
            jax.ShapeDtypeStruct((N, 1), jnp.float32),
        ],
    )(num_t, den_t, b1.reshape(1, H1 * C), W2,
      att_src2[0].reshape(C, 1), att_dst2[0].reshape(C, 1))
    return h2, a2s, a2d


def _k3_body(num_ref, den_ref, b2_ref, batch_ref, l1w_ref, l1b_ref,
             l2w_ref, l2b_ref, out_ref, pool_ref):
    i = pl.program_id(0)
    nblk = pl.num_programs(0)
    rows = num_ref.shape[0]

    @pl.when(i == 0)
    def _():
        pool_ref[...] = jnp.zeros_like(pool_ref)

    x2 = _elu(num_ref[...] / (den_ref[...] + 1e-16) + b2_ref[...])
    ext = lax.concatenate([x2, jnp.ones((rows, C), jnp.float32)], 1)
    onehot = (batch_ref[...] ==
              lax.broadcasted_iota(jnp.int32, (rows, G), 1).astype(jnp.float32)
              ).astype(jnp.float32)
    pool_ref[...] += lax.dot_general(
        onehot, ext, (((0,), (0,)), ((), ())),
        preferred_element_type=jnp.float32)

    @pl.when(i == nblk - 1)
    def _():
        pooled = pool_ref[...]
        mean = pooled[:, :C] / jnp.maximum(pooled[:, C:C + 1], 1.0)
        hid = jnp.maximum(
            jnp.dot(mean, l1w_ref[...], preferred_element_type=jnp.float32)
            + l1b_ref[...], 0.0)
        out_ref[...] = (jnp.dot(hid, l2w_ref[...],
                                preferred_element_type=jnp.float32)
                        + l2b_ref[...])


def _pool_mlp(num2, den2, b2, batch_f, lin1_w, lin1_b, lin2_w, lin2_b):
    nhid = lin1_w.shape[1]
    nout = lin2_w.shape[1]
    return pl.pallas_call(
        _k3_body,
        grid=(N // _ROWS,),
        in_specs=[
            pl.BlockSpec((_ROWS, C), lambda i: (i, 0)),
            pl.BlockSpec((_ROWS, 1), lambda i: (i, 0)),
            pl.BlockSpec((1, C), lambda i: (0, 0)),
            pl.BlockSpec((_ROWS, 1), lambda i: (i, 0)),
            pl.BlockSpec((C, nhid), lambda i: (0, 0)),
            pl.BlockSpec((1, nhid), lambda i: (0, 0)),
            pl.BlockSpec((nhid, nout), lambda i: (0, 0)),
            pl.BlockSpec((1, nout), lambda i: (0, 0)),
        ],
        out_specs=pl.BlockSpec((G, nout), lambda i: (0, 0)),
        out_shape=jax.ShapeDtypeStruct((G, nout), jnp.float32),
        scratch_shapes=[pltpu.VMEM((G, 2 * C), jnp.float32)],
    )(num2, den2, b2.reshape(1, C), batch_f, lin1_w,
      lin1_b.reshape(1, nhid), lin2_w, lin2_b.reshape(1, nout))


def kernel(x, edge_index, batch, W1, att_src1, att_dst1, b1,
           W2, att_src2, att_dst2, b2, lin1_w, lin1_b, lin2_w, lin2_b):
    src = edge_index[0]
    dst = edge_index[1]
    batch_f = batch.astype(jnp.float32).reshape(N, 1)

    h1, as1, ad1 = _proj_scores(x, W1, att_src1, att_dst1, H1, C)
    h1T = h1.T.reshape(H1 * C * N)
    as1t = as1.T.reshape(H1 * N)
    ad1t = ad1.T.reshape(H1 * N)
    njob1 = H1 * C // 4
    out1 = _sc_edge(h1T, as1t, ad1t, src, dst, 4, njob1, C // 4)
    o1 = out1.reshape(njob1, 5, N)
    num1_t = o1[:, :4, :].reshape(H1 * C, N).T
    den1_t = o1[::(C // 4), 4, :].T

    h2, a2s, a2d = _combine_l1(num1_t, den1_t, b1, W2, att_src2, att_dst2)

    h2T = h2.T.reshape(C * N)
    njob2 = C // 2
    out2 = _sc_edge(h2T, a2s.reshape(N), a2d.reshape(N), src, dst,
                    2, njob2, njob2)
    o2 = out2.reshape(njob2, 3, N)
    num2_t = o2[:, :2, :].reshape(C, N).T
    den2_t = o2[0, 2, :].reshape(N, 1)

    return _pool_mlp(num2_t, den2_t, b2, batch_f,
                     lin1_w, lin1_b, lin2_w, lin2_b)
